# Initial kernel scaffold; baseline (speedup 1.0000x reference)
import jax
import jax.numpy as jnp
from jax.experimental import pallas as pl


def kernel(memory, nodes, W1, b1, W2, b2):
    raise NotImplementedError("write your pallas kernel here")


# speedup vs baseline: 1.9282x; 1.9282x over previous
import functools

import jax
import jax.numpy as jnp
from jax import lax
from jax.experimental import pallas as pl
from jax.experimental.pallas import tpu as pltpu
from jax.experimental.pallas import tpu_sc as plsc

NC = 2
NS = 16
NW = NC * NS
CH = 128

_SC_PARAMS = pltpu.CompilerParams(use_tc_tiling_on_sc=False)
_MESH = dict(core_axis_name="c", subcore_axis_name="s")


def _sc_gather(mem_ref, nodes3d, n_chunks, D):
    mesh = plsc.VectorSubcoreMesh(**_MESH)

    @functools.partial(
        pl.kernel,
        mesh=mesh,
        out_type=jax.ShapeDtypeStruct((NW, n_chunks, CH, D), jnp.float32),
        scratch_types=[
            pltpu.VMEM((n_chunks, CH), jnp.int32),
            pltpu.VMEM((n_chunks, CH, D), jnp.float32),
            pltpu.SemaphoreType.DMA,
        ],
        compiler_params=_SC_PARAMS,
    )
    def k(idx_hbm, mem_hbm, out_hbm, idx_v, rows_v, sem):
        wid = lax.axis_index("s") * NC + lax.axis_index("c")
        pltpu.sync_copy(idx_hbm.at[wid], idx_v)
        copies = [
            pltpu.async_copy(mem_hbm.at[idx_v.at[j]], rows_v.at[j], sem)
            for j in range(n_chunks)
        ]
        for c in copies:
            c.wait()
        pltpu.sync_copy(rows_v, out_hbm.at[wid])

    return k(nodes3d, mem_ref)


def _sc_scatter(mem_ref, nodes3d, rows4d, n_chunks, D):
    mesh = plsc.VectorSubcoreMesh(**_MESH)

    @functools.partial(
        pl.kernel,
        mesh=mesh,
        out_type=(),
        scratch_types=[
            pltpu.VMEM((n_chunks, CH), jnp.int32),
            pltpu.VMEM((n_chunks, CH, D), jnp.float32),
            pltpu.SemaphoreType.DMA,
        ],
        compiler_params=_SC_PARAMS,
    )
    def k(idx_hbm, rows_hbm, mem_hbm, idx_v, rows_v, sem):
        wid = lax.axis_index("s") * NC + lax.axis_index("c")
        pltpu.sync_copy(idx_hbm.at[wid], idx_v)
        pltpu.sync_copy(rows_hbm.at[wid], rows_v)
        copies = [
            pltpu.async_copy(rows_v.at[j], mem_hbm.at[idx_v.at[j]], sem)
            for j in range(n_chunks)
        ]
        for c in copies:
            c.wait()

    k(nodes3d, rows4d, mem_ref)


def _tc_mlp(sel, W1, b1, W2, b2):
    B, D = sel.shape
    H = W1.shape[0]
    BLK = 2048

    def body(x_ref, w1_ref, b1_ref, w2_ref, b2_ref, o_ref):
        x = x_ref[...]
        h = lax.dot_general(
            x, w1_ref[...], (((1,), (1,)), ((), ())),
            preferred_element_type=jnp.float32,
        ) + b1_ref[...]
        h = jnp.where(h >= 0, h, 0.01 * h)
        o_ref[...] = lax.dot_general(
            h, w2_ref[...], (((1,), (1,)), ((), ())),
            preferred_element_type=jnp.float32,
        ) + b2_ref[...]

    return pl.pallas_call(
        body,
        out_shape=jax.ShapeDtypeStruct((B, D), jnp.float32),
        grid=(B // BLK,),
        in_specs=[
            pl.BlockSpec((BLK, D), lambda i: (i, 0)),
            pl.BlockSpec((H, D), lambda i: (0, 0)),
            pl.BlockSpec((1, H), lambda i: (0, 0)),
            pl.BlockSpec((D, H), lambda i: (0, 0)),
            pl.BlockSpec((1, D), lambda i: (0, 0)),
        ],
        out_specs=pl.BlockSpec((BLK, D), lambda i: (i, 0)),
    )(sel, W1, b1.reshape(1, H), W2, b2.reshape(1, D))


def kernel(memory, nodes, W1, b1, W2, b2):
    M, D = memory.shape
    B = nodes.shape[0]
    n_chunks = B // (NW * CH)
    nodes3d = nodes.reshape(NW, n_chunks, CH)

    mem_ref = jax.new_ref(memory)
    sel = _sc_gather(mem_ref, nodes3d, n_chunks, D).reshape(B, D)
    new_rows = _tc_mlp(sel, W1, b1, W2, b2)
    _sc_scatter(mem_ref, nodes3d, new_rows.reshape(NW, n_chunks, CH, D),
                n_chunks, D)
    return mem_ref[...]


# speedup vs baseline: 3.1648x; 1.6413x over previous
import functools

import jax
import jax.numpy as jnp
from jax import lax
from jax.experimental import pallas as pl
from jax.experimental.pallas import tpu as pltpu
from jax.experimental.pallas import tpu_sc as plsc

NC = 2
NS = 16
NW = NC * NS
D = 64
M = 1000000
B = 16384
CKW = 256
CAP = 768
SPAN = 31232
FULL_END = 999936
_MESH = dict(core_axis_name="c", subcore_axis_name="s")
_SC_PARAMS = pltpu.CompilerParams(needs_layout_passes=False)


def _prescan(nodes_v, matched_v, s_w, e_w):
    @pl.loop(0, B // 16, init_carry=jnp.zeros((16,), jnp.int32))
    def cnt_vec(g, cnt):
        v = nodes_v[pl.ds(g * 16, 16)]
        m = (v >= s_w) & (v < e_w)
        pos = cnt + plsc.cumsum(jnp.where(m, 1, 0).astype(jnp.int32)) - 1
        plsc.store_scatter(matched_v, [pos], v, mask=m & (pos < CAP))
        return cnt + plsc.all_reduce_population_count(m)

    return cnt_vec


def _for_matched(matched_v, cnt_vec, c0, width, fn):
    iota16 = lax.iota(jnp.int32, 16)

    @pl.loop(0, CAP // 16)
    def _(g):
        slots = g * 16 + iota16
        mv = matched_v[pl.ds(g * 16, 16)]
        inck = (slots < cnt_vec) & (mv >= c0) & (mv < c0 + width)

        @pl.when(jnp.any(inck))
        def _():
            p = mv - c0

            @pl.loop(0, D, unroll=8)
            def _(d):
                d_vec = jnp.broadcast_to(d, (16,))
                fn(d_vec, p, slots, inck)


def _sc_gather_sweep(memT, nodes):
    mesh = plsc.VectorSubcoreMesh(**_MESH)

    @functools.partial(
        pl.kernel,
        mesh=mesh,
        out_type=jax.ShapeDtypeStruct((D, NW * CAP), jnp.float32),
        scratch_types=[
            pltpu.VMEM((B,), jnp.int32),
            pltpu.VMEM((CAP,), jnp.int32),
            pltpu.VMEM((D, CKW), jnp.float32),
            pltpu.VMEM((D, CKW), jnp.float32),
            pltpu.VMEM((D, CAP), jnp.float32),
            pltpu.SemaphoreType.DMA,
            pltpu.SemaphoreType.DMA,
        ],
        compiler_params=_SC_PARAMS,
    )
    def k(memT_hbm, nodes_hbm, selB_hbm, nodes_v, matched_v, ch0, ch1,
          colbuf, sem0, sem1):
        wid = lax.axis_index("s") * NC + lax.axis_index("c")
        s_w = wid * SPAN
        e_w = jnp.where(wid == NW - 1, FULL_END, s_w + SPAN)
        nck = jnp.where(wid == NW - 1, (FULL_END - (NW - 1) * SPAN) // CKW,
                        SPAN // CKW)

        pltpu.sync_copy(nodes_hbm, nodes_v)
        cnt_vec = _prescan(nodes_v, matched_v, s_w, e_w)

        chs = (ch0, ch1)
        sems = (sem0, sem1)

        def c_of(kk):
            return pl.multiple_of(s_w + kk * CKW, 128)

        for b in range(2):
            pltpu.async_copy(
                memT_hbm.at[:, pl.ds(c_of(b), CKW)], chs[b], sems[b])

        def extract(ch):
            def fn(d_vec, p, slots, mask):
                vals = plsc.load_gather(ch, [d_vec, p], mask=mask)
                plsc.store_scatter(colbuf, [d_vec, slots], vals, mask=mask)
            return fn

        @pl.loop(0, (FULL_END - (NW - 1) * SPAN) // CKW // 2)
        def _(k2):
            for b in range(2):
                kk = 2 * k2 + b

                @pl.when(kk < nck)
                def _():
                    pltpu.make_async_copy(
                        memT_hbm.at[:, pl.ds(0, CKW)], chs[b], sems[b]
                    ).wait()
                    _for_matched(matched_v, cnt_vec, c_of(kk), CKW,
                                 extract(chs[b]))

                    @pl.when(kk + 2 < nck)
                    def _():
                        pltpu.async_copy(
                            memT_hbm.at[:, pl.ds(c_of(kk + 2), CKW)],
                            chs[b], sems[b])

        pltpu.sync_copy(colbuf, selB_hbm.at[:, pl.ds(wid * CAP, CAP)])

    return k(memT, nodes)


def _sc_merge_sweep(memT, nodes, outB):
    mesh = plsc.VectorSubcoreMesh(**_MESH)

    @functools.partial(
        pl.kernel,
        mesh=mesh,
        out_type=jax.ShapeDtypeStruct((D, M), jnp.float32),
        scratch_types=[
            pltpu.VMEM((B,), jnp.int32),
            pltpu.VMEM((CAP,), jnp.int32),
            pltpu.VMEM((D, CKW), jnp.float32),
            pltpu.VMEM((D, CKW), jnp.float32),
            pltpu.VMEM((D, CAP), jnp.float32),
            pltpu.SemaphoreType.DMA,
            pltpu.SemaphoreType.DMA,
            pltpu.SemaphoreType.DMA,
            pltpu.SemaphoreType.DMA,
        ],
        compiler_params=_SC_PARAMS,
    )
    def k(memT_hbm, nodes_hbm, outB_hbm, out_hbm, nodes_v, matched_v,
          ch0, ch1, colbuf, si0, si1, so0, so1):
        wid = lax.axis_index("s") * NC + lax.axis_index("c")
        s_w = wid * SPAN
        e_w = jnp.where(wid == NW - 1, FULL_END, s_w + SPAN)
        nck = jnp.where(wid == NW - 1, (FULL_END - (NW - 1) * SPAN) // CKW,
                        SPAN // CKW)

        pltpu.sync_copy(nodes_hbm, nodes_v)
        cnt_vec = _prescan(nodes_v, matched_v, s_w, e_w)
        pltpu.sync_copy(outB_hbm.at[:, pl.ds(wid * CAP, CAP)], colbuf)

        chs = (ch0, ch1)
        sin = (si0, si1)
        sout = (so0, so1)

        def c_of(kk):
            return pl.multiple_of(s_w + kk * CKW, 128)

        for b in range(2):
            pltpu.async_copy(
                memT_hbm.at[:, pl.ds(c_of(b), CKW)], chs[b], sin[b])

        def inject(ch):
            def fn(d_vec, p, slots, mask):
                vals = plsc.load_gather(colbuf, [d_vec, slots], mask=mask)
                plsc.store_scatter(ch, [d_vec, p], vals, mask=mask)
            return fn

        @pl.loop(0, (FULL_END - (NW - 1) * SPAN) // CKW // 2)
        def _(k2):
            for b in range(2):
                kk = 2 * k2 + b

                @pl.when(kk < nck)
                def _():
                    pltpu.make_async_copy(
                        memT_hbm.at[:, pl.ds(0, CKW)], chs[b], sin[b]
                    ).wait()
                    _for_matched(matched_v, cnt_vec, c_of(kk), CKW,
                                 inject(chs[b]))
                    pltpu.async_copy(
                        chs[b], out_hbm.at[:, pl.ds(c_of(kk), CKW)], sout[b])
                    pltpu.make_async_copy(
                        chs[b], out_hbm.at[:, pl.ds(0, CKW)], sout[b]
                    ).wait()

                    @pl.when(kk + 2 < nck)
                    def _():
                        pltpu.async_copy(
                            memT_hbm.at[:, pl.ds(c_of(kk + 2), CKW)],
                            chs[b], sin[b])

    return k(memT, nodes, outB)


def _tc_mlp_T(selB, W1, b1, W2, b2):
    Dn, N = selB.shape
    H = W1.shape[0]
    BLK = 2048

    def body(x_ref, w1_ref, b1_ref, w2_ref, b2_ref, o_ref):
        x = x_ref[...]
        h = lax.dot_general(
            w1_ref[...], x, (((1,), (0,)), ((), ())),
            preferred_element_type=jnp.float32,
        ) + b1_ref[...]
        h = jnp.where(h >= 0, h, 0.01 * h)
        o_ref[...] = lax.dot_general(
            w2_ref[...], h, (((1,), (0,)), ((), ())),
            preferred_element_type=jnp.float32,
        ) + b2_ref[...]

    return pl.pallas_call(
        body,
        out_shape=jax.ShapeDtypeStruct((Dn, N), jnp.float32),
        grid=(N // BLK,),
        in_specs=[
            pl.BlockSpec((Dn, BLK), lambda i: (0, i)),
            pl.BlockSpec((H, Dn), lambda i: (0, 0)),
            pl.BlockSpec((H, 1), lambda i: (0, 0)),
            pl.BlockSpec((Dn, H), lambda i: (0, 0)),
            pl.BlockSpec((Dn, 1), lambda i: (0, 0)),
        ],
        out_specs=pl.BlockSpec((Dn, BLK), lambda i: (0, i)),
    )(selB, W1, b1.reshape(H, 1), W2, b2.reshape(Dn, 1))


def _tc_tail_fix(out_full, memT, nodes128, W1, b1, W2, b2):
    TW = 128
    H = W1.shape[0]

    def body(o_alias, x_ref, n_ref, w1_ref, b1_ref, w2_ref, b2_ref, o_ref):
        del o_alias
        x = x_ref[...]
        h = lax.dot_general(
            w1_ref[...], x, (((1,), (0,)), ((), ())),
            preferred_element_type=jnp.float32,
        ) + b1_ref[...]
        h = jnp.where(h >= 0, h, 0.01 * h)
        o = lax.dot_general(
            w2_ref[...], h, (((1,), (0,)), ((), ())),
            preferred_element_type=jnp.float32,
        ) + b2_ref[...]
        nb = n_ref[...]
        iota_row = lax.broadcasted_iota(jnp.int32, (1, TW), 1)
        hit_row = jnp.zeros((1, TW), jnp.float32)
        for j in range(M - FULL_END):
            hj = jnp.where(jnp.any(nb == FULL_END + j), 1.0, 0.0)
            hit_row = jnp.where(iota_row == j, hj, hit_row)
        o_ref[...] = jnp.where(hit_row > 0, o, x)

    blk = FULL_END // TW
    return pl.pallas_call(
        body,
        out_shape=jax.ShapeDtypeStruct((D, M), jnp.float32),
        grid=(1,),
        in_specs=[
            pl.BlockSpec(memory_space=pl.ANY),
            pl.BlockSpec((D, TW), lambda i: (0, blk)),
            pl.BlockSpec((128, 128), lambda i: (0, 0)),
            pl.BlockSpec((H, D), lambda i: (0, 0)),
            pl.BlockSpec((H, 1), lambda i: (0, 0)),
            pl.BlockSpec((D, H), lambda i: (0, 0)),
            pl.BlockSpec((D, 1), lambda i: (0, 0)),
        ],
        out_specs=pl.BlockSpec((D, TW), lambda i: (0, blk)),
        input_output_aliases={0: 0},
    )(out_full, memT, nodes128, W1, b1.reshape(H, 1), W2, b2.reshape(D, 1))


def kernel(memory, nodes, W1, b1, W2, b2):
    memT = memory.T
    selB = _sc_gather_sweep(memT, nodes)
    outB = _tc_mlp_T(selB, W1, b1, W2, b2)
    out = _sc_merge_sweep(memT, nodes, outB)
    out = _tc_tail_fix(out, memT, nodes.reshape(128, 128), W1, b1, W2, b2)
    return out.T


# speedup vs baseline: 3.5409x; 1.1188x over previous
import functools

import jax
import jax.numpy as jnp
from jax import lax
from jax.experimental import pallas as pl
from jax.experimental.pallas import tpu as pltpu
from jax.experimental.pallas import tpu_sc as plsc

NC = 2
NS = 16
NW = NC * NS
D = 64
M = 1000000
B = 16384
CKW = 256
CAP = 768
SPAN = 31232
FULL_END = 999936
NPIECE = 8192
_MESH = dict(core_axis_name="c", subcore_axis_name="s")
_SC_PARAMS = pltpu.CompilerParams(needs_layout_passes=False)


def _prescan(nodes_hbm, nodes_v, matched_v, s_w, e_w):
    cnt0 = jnp.zeros((16,), jnp.int32)
    for piece in range(B // NPIECE):
        pltpu.sync_copy(nodes_hbm.at[pl.ds(piece * NPIECE, NPIECE)], nodes_v)

        @pl.loop(0, NPIECE // 16, init_carry=cnt0)
        def cnt0(g, cnt):
            v = nodes_v[pl.ds(g * 16, 16)]
            m = (v >= s_w) & (v < e_w)
            pos = cnt + plsc.cumsum(jnp.where(m, 1, 0).astype(jnp.int32)) - 1
            plsc.store_scatter(matched_v, [pos], v, mask=m & (pos < CAP))
            return cnt + plsc.all_reduce_population_count(m)

    return cnt0


def _for_matched(matched_v, cnt_vec, c0, width, fn):
    iota16 = lax.iota(jnp.int32, 16)

    @pl.loop(0, CAP // 16)
    def _(g):
        slots = g * 16 + iota16
        mv = matched_v[pl.ds(g * 16, 16)]
        inck = (slots < cnt_vec) & (mv >= c0) & (mv < c0 + width)

        @pl.when(jnp.any(inck))
        def _():
            p = mv - c0

            @pl.loop(0, D, unroll=8)
            def _(d):
                d_vec = jnp.broadcast_to(d, (16,))
                fn(d_vec, p, slots, inck)


def _sc_gather_sweep(memT, nodes):
    mesh = plsc.VectorSubcoreMesh(**_MESH)

    @functools.partial(
        pl.kernel,
        mesh=mesh,
        out_type=jax.ShapeDtypeStruct((D, NW * CAP), jnp.float32),
        scratch_types=[
            pltpu.VMEM((NPIECE,), jnp.int32),
            pltpu.VMEM((CAP,), jnp.int32),
            pltpu.VMEM((D, CKW), jnp.float32),
            pltpu.VMEM((D, CKW), jnp.float32),
            pltpu.VMEM((D, CKW), jnp.float32),
            pltpu.VMEM((D, CKW), jnp.float32),
            pltpu.VMEM((D, CAP), jnp.float32),
            pltpu.SemaphoreType.DMA,
            pltpu.SemaphoreType.DMA,
            pltpu.SemaphoreType.DMA,
            pltpu.SemaphoreType.DMA,
        ],
        compiler_params=_SC_PARAMS,
    )
    def k(memT_hbm, nodes_hbm, selB_hbm, nodes_v, matched_v, ch0, ch1,
          ch2, ch3, colbuf, sem0, sem1, sem2, sem3):
        wid = lax.axis_index("s") * NC + lax.axis_index("c")
        s_w = wid * SPAN
        e_w = jnp.where(wid == NW - 1, FULL_END, s_w + SPAN)
        nck = jnp.where(wid == NW - 1, (FULL_END - (NW - 1) * SPAN) // CKW,
                        SPAN // CKW)

        cnt_vec = _prescan(nodes_hbm, nodes_v, matched_v, s_w, e_w)

        chs = (ch0, ch1, ch2, ch3)
        sems = (sem0, sem1, sem2, sem3)

        def c_of(kk):
            return pl.multiple_of(s_w + kk * CKW, 128)

        for b in range(3):
            pltpu.async_copy(
                memT_hbm.at[:, pl.ds(c_of(b), CKW)], chs[b], sems[b])

        def extract(ch):
            def fn(d_vec, p, slots, mask):
                vals = plsc.load_gather(ch, [d_vec, p], mask=mask)
                plsc.store_scatter(colbuf, [d_vec, slots], vals, mask=mask)
            return fn

        @pl.loop(0, (FULL_END - (NW - 1) * SPAN) // CKW // 4)
        def _(k4):
            for b in range(4):
                kk = 4 * k4 + b

                @pl.when(kk < nck)
                def _():
                    pltpu.make_async_copy(
                        memT_hbm.at[:, pl.ds(0, CKW)], chs[b], sems[b]
                    ).wait()
                    _for_matched(matched_v, cnt_vec, c_of(kk), CKW,
                                 extract(chs[b]))

                    @pl.when(kk + 3 < nck)
                    def _():
                        pltpu.async_copy(
                            memT_hbm.at[:, pl.ds(c_of(kk + 3), CKW)],
                            chs[(b + 3) % 4], sems[(b + 3) % 4])

        pltpu.sync_copy(colbuf, selB_hbm.at[:, pl.ds(wid * CAP, CAP)])

    return k(memT, nodes)


def _sc_merge_sweep(memT, nodes, outB):
    mesh = plsc.VectorSubcoreMesh(**_MESH)

    @functools.partial(
        pl.kernel,
        mesh=mesh,
        out_type=jax.ShapeDtypeStruct((D, M), jnp.float32),
        scratch_types=[
            pltpu.VMEM((NPIECE,), jnp.int32),
            pltpu.VMEM((CAP,), jnp.int32),
            pltpu.VMEM((D, CKW), jnp.float32),
            pltpu.VMEM((D, CKW), jnp.float32),
            pltpu.VMEM((D, CKW), jnp.float32),
            pltpu.VMEM((D, CKW), jnp.float32),
            pltpu.VMEM((D, CAP), jnp.float32),
            pltpu.SemaphoreType.DMA,
            pltpu.SemaphoreType.DMA,
            pltpu.SemaphoreType.DMA,
            pltpu.SemaphoreType.DMA,
            pltpu.SemaphoreType.DMA,
            pltpu.SemaphoreType.DMA,
            pltpu.SemaphoreType.DMA,
            pltpu.SemaphoreType.DMA,
        ],
        compiler_params=_SC_PARAMS,
    )
    def k(memT_hbm, nodes_hbm, outB_hbm, out_hbm, nodes_v, matched_v,
          ch0, ch1, ch2, ch3, colbuf, si0, si1, si2, si3,
          so0, so1, so2, so3):
        wid = lax.axis_index("s") * NC + lax.axis_index("c")
        s_w = wid * SPAN
        e_w = jnp.where(wid == NW - 1, FULL_END, s_w + SPAN)
        nck = jnp.where(wid == NW - 1, (FULL_END - (NW - 1) * SPAN) // CKW,
                        SPAN // CKW)

        cnt_vec = _prescan(nodes_hbm, nodes_v, matched_v, s_w, e_w)
        pltpu.sync_copy(outB_hbm.at[:, pl.ds(wid * CAP, CAP)], colbuf)

        chs = (ch0, ch1, ch2, ch3)
        sin = (si0, si1, si2, si3)
        sout = (so0, so1, so2, so3)

        def c_of(kk):
            return pl.multiple_of(s_w + kk * CKW, 128)

        for b in range(2):
            pltpu.async_copy(
                memT_hbm.at[:, pl.ds(c_of(b), CKW)], chs[b], sin[b])

        def inject(ch):
            def fn(d_vec, p, slots, mask):
                vals = plsc.load_gather(colbuf, [d_vec, slots], mask=mask)
                plsc.store_scatter(ch, [d_vec, p], vals, mask=mask)
            return fn

        def wait_out(bb):
            pltpu.make_async_copy(
                chs[bb], out_hbm.at[:, pl.ds(0, CKW)], sout[bb]).wait()

        @pl.loop(0, (FULL_END - (NW - 1) * SPAN) // CKW // 4)
        def _(k4):
            for b in range(4):
                kk = 4 * k4 + b

                @pl.when(kk < nck)
                def _():
                    pltpu.make_async_copy(
                        memT_hbm.at[:, pl.ds(0, CKW)], chs[b], sin[b]
                    ).wait()
                    _for_matched(matched_v, cnt_vec, c_of(kk), CKW,
                                 inject(chs[b]))
                    pltpu.async_copy(
                        chs[b], out_hbm.at[:, pl.ds(c_of(kk), CKW)], sout[b])

                    @pl.when(kk + 2 < nck)
                    def _():
                        bn = (b + 2) % 4

                        @pl.when(kk >= 2)
                        def _():
                            wait_out(bn)

                        pltpu.async_copy(
                            memT_hbm.at[:, pl.ds(c_of(kk + 2), CKW)],
                            chs[bn], sin[bn])

        for off in (2, 1):
            for bb in range(4):
                @pl.when((nck - off) % 4 == bb)
                def _():
                    wait_out(bb)

    return k(memT, nodes, outB)


def _tc_mlp_T(selB, W1, b1, W2, b2):
    Dn, N = selB.shape
    H = W1.shape[0]
    BLK = 2048

    def body(x_ref, w1_ref, b1_ref, w2_ref, b2_ref, o_ref):
        x = x_ref[...]
        h = lax.dot_general(
            w1_ref[...], x, (((1,), (0,)), ((), ())),
            preferred_element_type=jnp.float32,
        ) + b1_ref[...]
        h = jnp.where(h >= 0, h, 0.01 * h)
        o_ref[...] = lax.dot_general(
            w2_ref[...], h, (((1,), (0,)), ((), ())),
            preferred_element_type=jnp.float32,
        ) + b2_ref[...]

    return pl.pallas_call(
        body,
        out_shape=jax.ShapeDtypeStruct((Dn, N), jnp.float32),
        grid=(N // BLK,),
        in_specs=[
            pl.BlockSpec((Dn, BLK), lambda i: (0, i)),
            pl.BlockSpec((H, Dn), lambda i: (0, 0)),
            pl.BlockSpec((H, 1), lambda i: (0, 0)),
            pl.BlockSpec((Dn, H), lambda i: (0, 0)),
            pl.BlockSpec((Dn, 1), lambda i: (0, 0)),
        ],
        out_specs=pl.BlockSpec((Dn, BLK), lambda i: (0, i)),
    )(selB, W1, b1.reshape(H, 1), W2, b2.reshape(Dn, 1))


def _tc_tail_fix(out_full, memT, nodes128, W1, b1, W2, b2):
    TW = 128
    H = W1.shape[0]

    def body(o_alias, x_ref, n_ref, w1_ref, b1_ref, w2_ref, b2_ref, o_ref):
        del o_alias
        x = x_ref[...]
        h = lax.dot_general(
            w1_ref[...], x, (((1,), (0,)), ((), ())),
            preferred_element_type=jnp.float32,
        ) + b1_ref[...]
        h = jnp.where(h >= 0, h, 0.01 * h)
        o = lax.dot_general(
            w2_ref[...], h, (((1,), (0,)), ((), ())),
            preferred_element_type=jnp.float32,
        ) + b2_ref[...]
        nb = n_ref[...]
        iota_row = lax.broadcasted_iota(jnp.int32, (1, TW), 1)
        hit_row = jnp.zeros((1, TW), jnp.float32)
        for j in range(M - FULL_END):
            hj = jnp.where(jnp.any(nb == FULL_END + j), 1.0, 0.0)
            hit_row = jnp.where(iota_row == j, hj, hit_row)
        o_ref[...] = jnp.where(hit_row > 0, o, x)

    blk = FULL_END // TW
    return pl.pallas_call(
        body,
        out_shape=jax.ShapeDtypeStruct((D, M), jnp.float32),
        grid=(1,),
        in_specs=[
            pl.BlockSpec(memory_space=pl.ANY),
            pl.BlockSpec((D, TW), lambda i: (0, blk)),
            pl.BlockSpec((128, 128), lambda i: (0, 0)),
            pl.BlockSpec((H, D), lambda i: (0, 0)),
            pl.BlockSpec((H, 1), lambda i: (0, 0)),
            pl.BlockSpec((D, H), lambda i: (0, 0)),
            pl.BlockSpec((D, 1), lambda i: (0, 0)),
        ],
        out_specs=pl.BlockSpec((D, TW), lambda i: (0, blk)),
        input_output_aliases={0: 0},
    )(out_full, memT, nodes128, W1, b1.reshape(H, 1), W2, b2.reshape(D, 1))


def kernel(memory, nodes, W1, b1, W2, b2):
    memT = memory.T
    selB = _sc_gather_sweep(memT, nodes)
    outB = _tc_mlp_T(selB, W1, b1, W2, b2)
    out = _sc_merge_sweep(memT, nodes, outB)
    out = _tc_tail_fix(out, memT, nodes.reshape(128, 128), W1, b1, W2, b2)
    return out.T


# speedup vs baseline: 6.0292x; 1.7027x over previous
import functools

import jax
import jax.numpy as jnp
from jax import lax
from jax.experimental import pallas as pl
from jax.experimental.pallas import tpu as pltpu
from jax.experimental.pallas import tpu_sc as plsc

NC = 2
NS = 16
NW = NC * NS
D = 64
M = 1000000
B = 16384
CKW = 256
CAP = 768
SPAN = 31232
FULL_END = 999936
NPIECE = 8192
_MESH = dict(core_axis_name="c", subcore_axis_name="s")
_SC_PARAMS = pltpu.CompilerParams(needs_layout_passes=False)


def _prescan(nodes_hbm, nodes_v, matched_v, s_w, e_w):
    cnt0 = jnp.zeros((16,), jnp.int32)
    for piece in range(B // NPIECE):
        pltpu.sync_copy(nodes_hbm.at[pl.ds(piece * NPIECE, NPIECE)], nodes_v)

        @pl.loop(0, NPIECE // 16, init_carry=cnt0)
        def cnt0(g, cnt):
            v = nodes_v[pl.ds(g * 16, 16)]
            m = (v >= s_w) & (v < e_w)
            pos = cnt + plsc.cumsum(jnp.where(m, 1, 0).astype(jnp.int32)) - 1
            plsc.store_scatter(matched_v, [pos], v, mask=m & (pos < CAP))
            return cnt + plsc.all_reduce_population_count(m)

    return cnt0


MAXM = 64


def _for_matched(matched_v, cnt_vec, px_v, sx_v, c0, width, fn):
    iota16 = lax.iota(jnp.int32, 16)

    @pl.loop(0, CAP // 16, init_carry=jnp.zeros((16,), jnp.int32))
    def cm_vec(g, cm):
        slots = g * 16 + iota16
        mv = matched_v[pl.ds(g * 16, 16)]
        inck = (slots < cnt_vec) & (mv >= c0) & (mv < c0 + width)
        pos = cm + plsc.cumsum(jnp.where(inck, 1, 0).astype(jnp.int32)) - 1
        ok = inck & (pos < MAXM)
        plsc.store_scatter(px_v, [pos], mv - c0, mask=ok)
        plsc.store_scatter(sx_v, [pos], slots, mask=ok)
        return cm + plsc.all_reduce_population_count(inck)

    @pl.loop(0, MAXM // 16)
    def _(gg):
        base = gg * 16

        @pl.when(base < cm_vec[0])
        def _():
            lanes = base + iota16
            mask = lanes < cm_vec
            p = px_v[pl.ds(base, 16)]
            s = sx_v[pl.ds(base, 16)]

            @pl.loop(0, D, unroll=8)
            def _(d):
                d_vec = jnp.broadcast_to(d, (16,))
                fn(d_vec, p, s, mask)


def _sc_gather_sweep(memT, nodes):
    mesh = plsc.VectorSubcoreMesh(**_MESH)

    @functools.partial(
        pl.kernel,
        mesh=mesh,
        out_type=jax.ShapeDtypeStruct((D, NW * CAP), jnp.float32),
        scratch_types=[
            pltpu.VMEM((NPIECE,), jnp.int32),
            pltpu.VMEM((CAP,), jnp.int32),
            pltpu.VMEM((D, CKW), jnp.float32),
            pltpu.VMEM((D, CKW), jnp.float32),
            pltpu.VMEM((D, CKW), jnp.float32),
            pltpu.VMEM((D, CKW), jnp.float32),
            pltpu.VMEM((D, CAP), jnp.float32),
            pltpu.VMEM((MAXM,), jnp.int32),
            pltpu.VMEM((MAXM,), jnp.int32),
            pltpu.SemaphoreType.DMA,
            pltpu.SemaphoreType.DMA,
            pltpu.SemaphoreType.DMA,
            pltpu.SemaphoreType.DMA,
        ],
        compiler_params=_SC_PARAMS,
    )
    def k(memT_hbm, nodes_hbm, selB_hbm, nodes_v, matched_v, ch0, ch1,
          ch2, ch3, colbuf, px_v, sx_v, sem0, sem1, sem2, sem3):
        wid = lax.axis_index("s") * NC + lax.axis_index("c")
        s_w = wid * SPAN
        e_w = jnp.where(wid == NW - 1, FULL_END, s_w + SPAN)
        nck = jnp.where(wid == NW - 1, (FULL_END - (NW - 1) * SPAN) // CKW,
                        SPAN // CKW)

        cnt_vec = _prescan(nodes_hbm, nodes_v, matched_v, s_w, e_w)

        chs = (ch0, ch1, ch2, ch3)
        sems = (sem0, sem1, sem2, sem3)

        def c_of(kk):
            return pl.multiple_of(s_w + kk * CKW, 128)

        for b in range(3):
            pltpu.async_copy(
                memT_hbm.at[:, pl.ds(c_of(b), CKW)], chs[b], sems[b])

        def extract(ch):
            def fn(d_vec, p, slots, mask):
                vals = plsc.load_gather(ch, [d_vec, p], mask=mask)
                plsc.store_scatter(colbuf, [d_vec, slots], vals, mask=mask)
            return fn

        @pl.loop(0, (FULL_END - (NW - 1) * SPAN) // CKW // 4)
        def _(k4):
            for b in range(4):
                kk = 4 * k4 + b

                @pl.when(kk < nck)
                def _():
                    pltpu.make_async_copy(
                        memT_hbm.at[:, pl.ds(0, CKW)], chs[b], sems[b]
                    ).wait()
                    _for_matched(matched_v, cnt_vec, px_v, sx_v,
                                 c_of(kk), CKW, extract(chs[b]))

                    @pl.when(kk + 3 < nck)
                    def _():
                        pltpu.async_copy(
                            memT_hbm.at[:, pl.ds(c_of(kk + 3), CKW)],
                            chs[(b + 3) % 4], sems[(b + 3) % 4])

        pltpu.sync_copy(colbuf, selB_hbm.at[:, pl.ds(wid * CAP, CAP)])

    return k(memT, nodes)


def _sc_merge_sweep(memT, nodes, outB):
    mesh = plsc.VectorSubcoreMesh(**_MESH)

    @functools.partial(
        pl.kernel,
        mesh=mesh,
        out_type=jax.ShapeDtypeStruct((D, M), jnp.float32),
        scratch_types=[
            pltpu.VMEM((NPIECE,), jnp.int32),
            pltpu.VMEM((CAP,), jnp.int32),
            pltpu.VMEM((D, CKW), jnp.float32),
            pltpu.VMEM((D, CKW), jnp.float32),
            pltpu.VMEM((D, CKW), jnp.float32),
            pltpu.VMEM((D, CKW), jnp.float32),
            pltpu.VMEM((D, CAP), jnp.float32),
            pltpu.VMEM((MAXM,), jnp.int32),
            pltpu.VMEM((MAXM,), jnp.int32),
            pltpu.SemaphoreType.DMA,
            pltpu.SemaphoreType.DMA,
            pltpu.SemaphoreType.DMA,
            pltpu.SemaphoreType.DMA,
            pltpu.SemaphoreType.DMA,
            pltpu.SemaphoreType.DMA,
            pltpu.SemaphoreType.DMA,
            pltpu.SemaphoreType.DMA,
        ],
        compiler_params=_SC_PARAMS,
    )
    def k(memT_hbm, nodes_hbm, outB_hbm, out_hbm, nodes_v, matched_v,
          ch0, ch1, ch2, ch3, colbuf, px_v, sx_v, si0, si1, si2, si3,
          so0, so1, so2, so3):
        wid = lax.axis_index("s") * NC + lax.axis_index("c")
        s_w = wid * SPAN
        e_w = jnp.where(wid == NW - 1, FULL_END, s_w + SPAN)
        nck = jnp.where(wid == NW - 1, (FULL_END - (NW - 1) * SPAN) // CKW,
                        SPAN // CKW)

        cnt_vec = _prescan(nodes_hbm, nodes_v, matched_v, s_w, e_w)
        pltpu.sync_copy(outB_hbm.at[:, pl.ds(wid * CAP, CAP)], colbuf)

        chs = (ch0, ch1, ch2, ch3)
        sin = (si0, si1, si2, si3)
        sout = (so0, so1, so2, so3)

        def c_of(kk):
            return pl.multiple_of(s_w + kk * CKW, 128)

        for b in range(2):
            pltpu.async_copy(
                memT_hbm.at[:, pl.ds(c_of(b), CKW)], chs[b], sin[b])

        def inject(ch):
            def fn(d_vec, p, slots, mask):
                vals = plsc.load_gather(colbuf, [d_vec, slots], mask=mask)
                plsc.store_scatter(ch, [d_vec, p], vals, mask=mask)
            return fn

        def wait_out(bb):
            pltpu.make_async_copy(
                chs[bb], out_hbm.at[:, pl.ds(0, CKW)], sout[bb]).wait()

        @pl.loop(0, (FULL_END - (NW - 1) * SPAN) // CKW // 4)
        def _(k4):
            for b in range(4):
                kk = 4 * k4 + b

                @pl.when(kk < nck)
                def _():
                    pltpu.make_async_copy(
                        memT_hbm.at[:, pl.ds(0, CKW)], chs[b], sin[b]
                    ).wait()
                    _for_matched(matched_v, cnt_vec, px_v, sx_v,
                                 c_of(kk), CKW, inject(chs[b]))
                    pltpu.async_copy(
                        chs[b], out_hbm.at[:, pl.ds(c_of(kk), CKW)], sout[b])

                    @pl.when(kk + 2 < nck)
                    def _():
                        bn = (b + 2) % 4

                        @pl.when(kk >= 2)
                        def _():
                            wait_out(bn)

                        pltpu.async_copy(
                            memT_hbm.at[:, pl.ds(c_of(kk + 2), CKW)],
                            chs[bn], sin[bn])

        for off in (2, 1):
            for bb in range(4):
                @pl.when((nck - off) % 4 == bb)
                def _():
                    wait_out(bb)

    return k(memT, nodes, outB)


def _tc_mlp_T(selB, W1, b1, W2, b2):
    Dn, N = selB.shape
    H = W1.shape[0]
    BLK = 2048

    def body(x_ref, w1_ref, b1_ref, w2_ref, b2_ref, o_ref):
        x = x_ref[...]
        h = lax.dot_general(
            w1_ref[...], x, (((1,), (0,)), ((), ())),
            preferred_element_type=jnp.float32,
        ) + b1_ref[...]
        h = jnp.where(h >= 0, h, 0.01 * h)
        o_ref[...] = lax.dot_general(
            w2_ref[...], h, (((1,), (0,)), ((), ())),
            preferred_element_type=jnp.float32,
        ) + b2_ref[...]

    return pl.pallas_call(
        body,
        out_shape=jax.ShapeDtypeStruct((Dn, N), jnp.float32),
        grid=(N // BLK,),
        in_specs=[
            pl.BlockSpec((Dn, BLK), lambda i: (0, i)),
            pl.BlockSpec((H, Dn), lambda i: (0, 0)),
            pl.BlockSpec((H, 1), lambda i: (0, 0)),
            pl.BlockSpec((Dn, H), lambda i: (0, 0)),
            pl.BlockSpec((Dn, 1), lambda i: (0, 0)),
        ],
        out_specs=pl.BlockSpec((Dn, BLK), lambda i: (0, i)),
    )(selB, W1, b1.reshape(H, 1), W2, b2.reshape(Dn, 1))


def _tc_tail_fix(out_full, memT, nodes128, W1, b1, W2, b2):
    TW = 128
    H = W1.shape[0]

    def body(o_alias, x_ref, n_ref, w1_ref, b1_ref, w2_ref, b2_ref, o_ref):
        del o_alias
        x = x_ref[...]
        h = lax.dot_general(
            w1_ref[...], x, (((1,), (0,)), ((), ())),
            preferred_element_type=jnp.float32,
        ) + b1_ref[...]
        h = jnp.where(h >= 0, h, 0.01 * h)
        o = lax.dot_general(
            w2_ref[...], h, (((1,), (0,)), ((), ())),
            preferred_element_type=jnp.float32,
        ) + b2_ref[...]
        nb = n_ref[...]
        iota_row = lax.broadcasted_iota(jnp.int32, (1, TW), 1)
        hit_row = jnp.zeros((1, TW), jnp.float32)
        for j in range(M - FULL_END):
            hj = jnp.where(jnp.any(nb == FULL_END + j), 1.0, 0.0)
            hit_row = jnp.where(iota_row == j, hj, hit_row)
        o_ref[...] = jnp.where(hit_row > 0, o, x)

    blk = FULL_END // TW
    return pl.pallas_call(
        body,
        out_shape=jax.ShapeDtypeStruct((D, M), jnp.float32),
        grid=(1,),
        in_specs=[
            pl.BlockSpec(memory_space=pl.ANY),
            pl.BlockSpec((D, TW), lambda i: (0, blk)),
            pl.BlockSpec((128, 128), lambda i: (0, 0)),
            pl.BlockSpec((H, D), lambda i: (0, 0)),
            pl.BlockSpec((H, 1), lambda i: (0, 0)),
            pl.BlockSpec((D, H), lambda i: (0, 0)),
            pl.BlockSpec((D, 1), lambda i: (0, 0)),
        ],
        out_specs=pl.BlockSpec((D, TW), lambda i: (0, blk)),
        input_output_aliases={0: 0},
    )(out_full, memT, nodes128, W1, b1.reshape(H, 1), W2, b2.reshape(D, 1))


def kernel(memory, nodes, W1, b1, W2, b2):
    memT = memory.T
    selB = _sc_gather_sweep(memT, nodes)
    outB = _tc_mlp_T(selB, W1, b1, W2, b2)
    out = _sc_merge_sweep(memT, nodes, outB)
    out = _tc_tail_fix(out, memT, nodes.reshape(128, 128), W1, b1, W2, b2)
    return out.T
